# Initial kernel scaffold; baseline (speedup 1.0000x reference)
import jax
import jax.numpy as jnp
from jax.experimental import pallas as pl


def kernel(x, adj, W1, b1, W2, b2, W3, b3, Wlin, blin):
    raise NotImplementedError("write your pallas kernel here")


# speedup vs baseline: 22.5297x; 22.5297x over previous
import functools

import jax
import jax.numpy as jnp
from jax import lax
from jax.experimental import pallas as pl
from jax.experimental.pallas import tpu as pltpu

_T = 10
_K = 2
_SUB = 16
_LANE = 128

_HIGH = lax.Precision.HIGHEST


def _dilate_kernel(m0_ref, keep_ref):
    keep_ref[...] = m0_ref[...]
    n_rows = m0_ref.shape[1]
    lanes_per_row = _SUB * _LANE

    iota_h = lax.broadcasted_iota(jnp.int32, (_SUB, _LANE), 0)
    iota_l = lax.broadcasted_iota(jnp.int32, (_SUB, _LANE), 1)
    flat_iota = iota_h * _LANE + iota_l

    def body(n, carry):
        row = keep_ref[0, pl.ds(n, 1)].reshape(_SUB, _LANE)
        m32 = row.astype(jnp.int32)

        cs = m32
        for sh in (1, 2, 4, 8, 16, 32, 64):
            cs = cs + jnp.concatenate(
                [jnp.zeros((_SUB, sh), jnp.int32), cs[:, : _LANE - sh]], axis=1
            )
        rowtot = cs[:, _LANE - 1 :]
        ps = rowtot
        for sh in (1, 2, 4, 8):
            ps = ps + jnp.concatenate(
                [jnp.zeros((sh, 1), jnp.int32), ps[: _SUB - sh]], axis=0
            )
        m = ps[_SUB - 1, 0]
        cs = cs + (ps - rowtot)

        maskb = m32 > 0
        sf = jnp.where(m > _T, (m + (_K - 1)) // _K, 1)
        removable = m > 1

        rem = maskb & removable & (cs % sf == 0)
        keep_ref[0, pl.ds(n, 1)] = jnp.where(
            rem, jnp.int8(0), row
        ).reshape(1, _SUB, _LANE)

        n_hi = n // _LANE
        n_lo = n % _LANE
        col_hit = (iota_h == n_hi) & (iota_l == n_lo)

        def zero_col_at_rank(t):
            eq = maskb & (cs == t)
            c = jnp.max(jnp.where(eq, flat_iota, -1))

            @pl.when(c >= 0)
            def _():
                rc = keep_ref[0, pl.ds(c, 1)].reshape(_SUB, _LANE)
                keep_ref[0, pl.ds(c, 1)] = jnp.where(
                    col_hit, jnp.int8(0), rc
                ).reshape(1, _SUB, _LANE)

        for r in (1, 2):

            @pl.when(removable & (r * sf <= m))
            def _(r=r):
                zero_col_at_rank(r * sf)

        @pl.when(removable & (m <= _T))
        def _():
            for r in range(3, _T + 1):

                @pl.when(r <= m)
                def _(r=r):
                    zero_col_at_rank(r)

        return carry

    lax.fori_loop(0, n_rows, body, 0, unroll=False)


def _sage_kernel(adj_ref, keep_ref, h_ref, w_ref, b_ref, out_ref):
    ad = adj_ref[0] * keep_ref[0].astype(jnp.float32)
    t = jnp.dot(ad, h_ref[0], precision=_HIGH)
    deg = jnp.maximum(jnp.sum(ad, axis=1, keepdims=True), 1.0)
    t = t / deg
    y = jnp.dot(t, w_ref[...], precision=_HIGH) + b_ref[...]
    nrm = jnp.sqrt(jnp.sum(y * y, axis=1, keepdims=True))
    y = y / jnp.maximum(nrm, 1e-12)
    out_ref[0] = jnp.maximum(y, 0.0)


def _final_kernel(x1_ref, x2_ref, x3_ref, wt_ref, b_ref, out_ref):
    xc = jnp.concatenate([x1_ref[0], x2_ref[0], x3_ref[0]], axis=1)
    out_ref[0] = jnp.dot(xc, wt_ref[...], precision=_HIGH) + b_ref[...]


def _dilate(mask0):
    b, n = mask0.shape[0], mask0.shape[1]
    m4 = mask0.reshape(b, n, _SUB, _LANE)
    keep = pl.pallas_call(
        _dilate_kernel,
        grid=(b,),
        in_specs=[
            pl.BlockSpec((1, n, _SUB, _LANE), lambda i: (i, 0, 0, 0)),
        ],
        out_specs=pl.BlockSpec((1, n, _SUB, _LANE), lambda i: (i, 0, 0, 0)),
        out_shape=jax.ShapeDtypeStruct((b, n, _SUB, _LANE), jnp.int8),
        compiler_params=pltpu.CompilerParams(
            dimension_semantics=("arbitrary",),
        ),
    )(m4)
    return keep.reshape(b, n, n)


def _sage_layer(adj, keep, h, w, bias, tile_m):
    b, n, _ = adj.shape
    dh = w.shape[1]
    return pl.pallas_call(
        _sage_kernel,
        grid=(b, n // tile_m),
        in_specs=[
            pl.BlockSpec((1, tile_m, n), lambda i, j: (i, j, 0)),
            pl.BlockSpec((1, tile_m, n), lambda i, j: (i, j, 0)),
            pl.BlockSpec((1, n, h.shape[2]), lambda i, j: (i, 0, 0)),
            pl.BlockSpec(w.shape, lambda i, j: (0, 0)),
            pl.BlockSpec((1, dh), lambda i, j: (0, 0)),
        ],
        out_specs=pl.BlockSpec((1, tile_m, dh), lambda i, j: (i, j, 0)),
        out_shape=jax.ShapeDtypeStruct((b, n, dh), jnp.float32),
        compiler_params=pltpu.CompilerParams(
            dimension_semantics=("parallel", "parallel"),
        ),
    )(adj, keep, h, w, bias)


def _final_linear(x1, x2, x3, wlin, blin):
    b, n, e = x1.shape
    wt = wlin.T
    bias = blin.reshape(1, -1)
    return pl.pallas_call(
        _final_kernel,
        grid=(b,),
        in_specs=[
            pl.BlockSpec((1, n, e), lambda i: (i, 0, 0)),
            pl.BlockSpec((1, n, e), lambda i: (i, 0, 0)),
            pl.BlockSpec((1, n, e), lambda i: (i, 0, 0)),
            pl.BlockSpec(wt.shape, lambda i: (0, 0)),
            pl.BlockSpec((1, bias.shape[1]), lambda i: (0, 0)),
        ],
        out_specs=pl.BlockSpec((1, n, bias.shape[1]), lambda i: (i, 0, 0)),
        out_shape=jax.ShapeDtypeStruct((b, n, bias.shape[1]), jnp.float32),
        compiler_params=pltpu.CompilerParams(
            dimension_semantics=("parallel",),
        ),
    )(x1, x2, x3, wt, bias)


@jax.jit
def kernel(x, adj, W1, b1, W2, b2, W3, b3, Wlin, blin):
    b, n, _ = x.shape
    mask0 = (adj > 0).astype(jnp.int8)
    keep = _dilate(mask0)
    tile_m = 512
    x1 = _sage_layer(adj, keep, x, W1, b1.reshape(1, -1), tile_m)
    x2 = _sage_layer(adj, keep, x1, W2, b2.reshape(1, -1), tile_m)
    x3 = _sage_layer(adj, keep, x2, W3, b3.reshape(1, -1), tile_m)
    return _final_linear(x1, x2, x3, Wlin, blin)


# speedup vs baseline: 27.8047x; 1.2341x over previous
import functools

import jax
import jax.numpy as jnp
from jax import lax
from jax.experimental import pallas as pl
from jax.experimental.pallas import tpu as pltpu

_T = 10
_K = 2
_SUB = 16
_LANE = 128

_HIGH = lax.Precision.HIGHEST


def _dilate_kernel(m0_ref, keep_ref):
    keep_ref[...] = m0_ref[...].astype(jnp.int32)
    n_rows = m0_ref.shape[1]

    iota_h = lax.broadcasted_iota(jnp.int32, (_SUB, _LANE), 0)
    iota_l = lax.broadcasted_iota(jnp.int32, (_SUB, _LANE), 1)
    flat_iota = iota_h * _LANE + iota_l

    def body(n, carry):
        row = keep_ref[0, pl.ds(n, 1)].reshape(_SUB, _LANE)

        cs = row
        for sh in (1, 2, 4, 8, 16, 32, 64):
            cs = cs + jnp.concatenate(
                [jnp.zeros((_SUB, sh), jnp.int32), cs[:, : _LANE - sh]], axis=1
            )
        rowtot = cs[:, _LANE - 1 :]
        ps = rowtot
        for sh in (1, 2, 4, 8):
            ps = ps + jnp.concatenate(
                [jnp.zeros((sh, 1), jnp.int32), ps[: _SUB - sh]], axis=0
            )
        m = ps[_SUB - 1, 0]
        cs = cs + (ps - rowtot)

        maskb = row > 0
        sf = jnp.where(m > _T, (m + (_K - 1)) // _K, 1)
        removable = m > 1

        rem = maskb & removable & (cs % sf == 0)
        keep_ref[0, pl.ds(n, 1)] = jnp.where(rem, 0, row).reshape(
            1, _SUB, _LANE
        )

        n_hi = n // _LANE
        n_lo = n % _LANE
        col_keep = 1 - ((iota_h == n_hi) & (iota_l == n_lo)).astype(jnp.int32)

        def col_index_of_rank(t):
            return jnp.max(jnp.where(maskb & (cs == t), flat_iota, -1))

        def zero_col(c):
            rc = keep_ref[0, pl.ds(c, 1)].reshape(_SUB, _LANE)
            keep_ref[0, pl.ds(c, 1)] = (rc * col_keep).reshape(1, _SUB, _LANE)

        c1 = col_index_of_rank(sf)
        c2 = col_index_of_rank(2 * sf)

        @pl.when(removable & (sf <= m))
        def _():
            zero_col(c1)

        @pl.when(removable & (2 * sf <= m))
        def _():
            zero_col(c2)

        @pl.when(removable & (m <= _T))
        def _():
            cr = [col_index_of_rank(r) for r in range(3, _T + 1)]
            for i, r in enumerate(range(3, _T + 1)):

                @pl.when(r <= m)
                def _(i=i):
                    zero_col(cr[i])

        return carry

    lax.fori_loop(0, n_rows, body, 0, unroll=False)


def _sage_kernel(adj_ref, keep_ref, h_ref, w_ref, b_ref, out_ref):
    ad = adj_ref[0] * keep_ref[0].astype(jnp.float32)
    t = jnp.dot(ad, h_ref[0], precision=_HIGH)
    deg = jnp.maximum(jnp.sum(ad, axis=1, keepdims=True), 1.0)
    t = t / deg
    y = jnp.dot(t, w_ref[...], precision=_HIGH) + b_ref[...]
    nrm = jnp.sqrt(jnp.sum(y * y, axis=1, keepdims=True))
    y = y / jnp.maximum(nrm, 1e-12)
    out_ref[0] = jnp.maximum(y, 0.0)


def _final_kernel(x1_ref, x2_ref, x3_ref, wt_ref, b_ref, out_ref):
    xc = jnp.concatenate([x1_ref[0], x2_ref[0], x3_ref[0]], axis=1)
    out_ref[0] = jnp.dot(xc, wt_ref[...], precision=_HIGH) + b_ref[...]


def _dilate(mask0):
    b, n = mask0.shape[0], mask0.shape[1]
    m4 = mask0.reshape(b, n, _SUB, _LANE)
    keep = pl.pallas_call(
        _dilate_kernel,
        grid=(b,),
        in_specs=[
            pl.BlockSpec((1, n, _SUB, _LANE), lambda i: (i, 0, 0, 0)),
        ],
        out_specs=pl.BlockSpec((1, n, _SUB, _LANE), lambda i: (i, 0, 0, 0)),
        out_shape=jax.ShapeDtypeStruct((b, n, _SUB, _LANE), jnp.int32),
        compiler_params=pltpu.CompilerParams(
            dimension_semantics=("arbitrary",),
        ),
    )(m4)
    return keep.reshape(b, n, n)


def _sage_layer(adj, keep, h, w, bias, tile_m):
    b, n, _ = adj.shape
    dh = w.shape[1]
    return pl.pallas_call(
        _sage_kernel,
        grid=(b, n // tile_m),
        in_specs=[
            pl.BlockSpec((1, tile_m, n), lambda i, j: (i, j, 0)),
            pl.BlockSpec((1, tile_m, n), lambda i, j: (i, j, 0)),
            pl.BlockSpec((1, n, h.shape[2]), lambda i, j: (i, 0, 0)),
            pl.BlockSpec(w.shape, lambda i, j: (0, 0)),
            pl.BlockSpec((1, dh), lambda i, j: (0, 0)),
        ],
        out_specs=pl.BlockSpec((1, tile_m, dh), lambda i, j: (i, j, 0)),
        out_shape=jax.ShapeDtypeStruct((b, n, dh), jnp.float32),
        compiler_params=pltpu.CompilerParams(
            dimension_semantics=("parallel", "parallel"),
        ),
    )(adj, keep, h, w, bias)


def _final_linear(x1, x2, x3, wlin, blin):
    b, n, e = x1.shape
    wt = wlin.T
    bias = blin.reshape(1, -1)
    return pl.pallas_call(
        _final_kernel,
        grid=(b,),
        in_specs=[
            pl.BlockSpec((1, n, e), lambda i: (i, 0, 0)),
            pl.BlockSpec((1, n, e), lambda i: (i, 0, 0)),
            pl.BlockSpec((1, n, e), lambda i: (i, 0, 0)),
            pl.BlockSpec(wt.shape, lambda i: (0, 0)),
            pl.BlockSpec((1, bias.shape[1]), lambda i: (0, 0)),
        ],
        out_specs=pl.BlockSpec((1, n, bias.shape[1]), lambda i: (i, 0, 0)),
        out_shape=jax.ShapeDtypeStruct((b, n, bias.shape[1]), jnp.float32),
        compiler_params=pltpu.CompilerParams(
            dimension_semantics=("parallel",),
        ),
    )(x1, x2, x3, wt, bias)


@jax.jit
def kernel(x, adj, W1, b1, W2, b2, W3, b3, Wlin, blin):
    b, n, _ = x.shape
    mask0 = (adj > 0).astype(jnp.int8)
    keep = _dilate(mask0)
    tile_m = 512
    x1 = _sage_layer(adj, keep, x, W1, b1.reshape(1, -1), tile_m)
    x2 = _sage_layer(adj, keep, x1, W2, b2.reshape(1, -1), tile_m)
    x3 = _sage_layer(adj, keep, x2, W3, b3.reshape(1, -1), tile_m)
    return _final_linear(x1, x2, x3, Wlin, blin)


# speedup vs baseline: 50.4702x; 1.8152x over previous
import functools

import jax
import jax.numpy as jnp
from jax import lax
from jax.experimental import pallas as pl
from jax.experimental.pallas import tpu as pltpu

_T = 10
_K = 2
_SUB = 16
_LANE = 128

_HIGH = lax.Precision.HIGHEST


def _dilate_kernel(m0_ref, keep_ref):
    keep_ref[...] = m0_ref[...].astype(jnp.float32)
    n_rows = m0_ref.shape[1]

    iota_h = lax.broadcasted_iota(jnp.int32, (_SUB, _LANE), 0)
    iota_l = lax.broadcasted_iota(jnp.int32, (_SUB, _LANE), 1)
    flat_iota = iota_h * _LANE + iota_l
    U = (
        lax.broadcasted_iota(jnp.int32, (_LANE, _LANE), 0)
        <= lax.broadcasted_iota(jnp.int32, (_LANE, _LANE), 1)
    ).astype(jnp.float32)
    Ls = (
        lax.broadcasted_iota(jnp.int32, (_SUB, _SUB), 1)
        < lax.broadcasted_iota(jnp.int32, (_SUB, _SUB), 0)
    ).astype(jnp.float32)

    def apply_fix(c, col):
        ck = 1.0 - (
            (iota_h == col // _LANE) & (iota_l == col % _LANE)
        ).astype(jnp.float32)
        rc = keep_ref[0, pl.ds(c, 1)].reshape(_SUB, _LANE)
        keep_ref[0, pl.ds(c, 1)] = (rc * ck).reshape(1, _SUB, _LANE)

    def body(n, carry):
        a1, c1p, a2, c2p = carry

        @pl.when(a1 == 1)
        def _():
            apply_fix(c1p, n - 1)

        @pl.when(a2 == 1)
        def _():
            apply_fix(c2p, n - 1)

        row = keep_ref[0, pl.ds(n, 1)].reshape(_SUB, _LANE)
        pc = jnp.dot(row, U)
        rs = pc[:, _LANE - 1 :]
        excl = jnp.dot(Ls, rs)
        cs = pc + excl

        m = cs[_SUB - 1, _LANE - 1].astype(jnp.int32)
        maskb = row > 0.0
        sf = jnp.where(m > _T, (m + (_K - 1)) // _K, 1)
        removable = m > 1
        small = m <= _T
        t1 = sf.astype(jnp.float32)
        t2 = (2 * sf).astype(jnp.float32)

        rem = maskb & removable & ((cs == t1) | (cs == t2) | small)
        keep_ref[0, pl.ds(n, 1)] = jnp.where(rem, 0.0, row).reshape(
            1, _SUB, _LANE
        )

        c1 = jnp.max(jnp.where(maskb & (cs == t1), flat_iota, -1))
        c2 = jnp.max(jnp.where(maskb & (cs == t2), flat_iota, -1))
        a1n = (removable & (sf <= m)).astype(jnp.int32)
        a2n = (removable & (2 * sf <= m)).astype(jnp.int32)

        @pl.when(removable & small & (m >= 3))
        def _():
            crs = [
                jnp.max(jnp.where(maskb & (cs == float(r)), flat_iota, -1))
                for r in range(3, _T + 1)
            ]
            for i, r in enumerate(range(3, _T + 1)):

                @pl.when(r <= m)
                def _(i=i):
                    apply_fix(crs[i], n)

        return (a1n, c1, a2n, c2)

    z = jnp.int32(0)
    a1, c1, a2, c2 = lax.fori_loop(0, n_rows, body, (z, z, z, z))

    @pl.when(a1 == 1)
    def _():
        apply_fix(c1, n_rows - 1)

    @pl.when(a2 == 1)
    def _():
        apply_fix(c2, n_rows - 1)


def _sage_kernel(adj_ref, keep_ref, h_ref, w_ref, b_ref, out_ref):
    ad = adj_ref[0] * keep_ref[0]
    t = jnp.dot(ad, h_ref[0], precision=_HIGH)
    deg = jnp.maximum(jnp.sum(ad, axis=1, keepdims=True), 1.0)
    t = t / deg
    y = jnp.dot(t, w_ref[...], precision=_HIGH) + b_ref[...]
    nrm = jnp.sqrt(jnp.sum(y * y, axis=1, keepdims=True))
    y = y / jnp.maximum(nrm, 1e-12)
    out_ref[0] = jnp.maximum(y, 0.0)


def _final_kernel(x1_ref, x2_ref, x3_ref, wt_ref, b_ref, out_ref):
    xc = jnp.concatenate([x1_ref[0], x2_ref[0], x3_ref[0]], axis=1)
    out_ref[0] = jnp.dot(xc, wt_ref[...], precision=_HIGH) + b_ref[...]


def _dilate(mask0):
    b, n = mask0.shape[0], mask0.shape[1]
    m4 = mask0.reshape(b, n, _SUB, _LANE)
    keep = pl.pallas_call(
        _dilate_kernel,
        grid=(b,),
        in_specs=[
            pl.BlockSpec((1, n, _SUB, _LANE), lambda i: (i, 0, 0, 0)),
        ],
        out_specs=pl.BlockSpec((1, n, _SUB, _LANE), lambda i: (i, 0, 0, 0)),
        out_shape=jax.ShapeDtypeStruct((b, n, _SUB, _LANE), jnp.float32),
        compiler_params=pltpu.CompilerParams(
            dimension_semantics=("arbitrary",),
        ),
    )(m4)
    return keep.reshape(b, n, n)


def _sage_layer(adj, keep, h, w, bias, tile_m):
    b, n, _ = adj.shape
    dh = w.shape[1]
    return pl.pallas_call(
        _sage_kernel,
        grid=(b, n // tile_m),
        in_specs=[
            pl.BlockSpec((1, tile_m, n), lambda i, j: (i, j, 0)),
            pl.BlockSpec((1, tile_m, n), lambda i, j: (i, j, 0)),
            pl.BlockSpec((1, n, h.shape[2]), lambda i, j: (i, 0, 0)),
            pl.BlockSpec(w.shape, lambda i, j: (0, 0)),
            pl.BlockSpec((1, dh), lambda i, j: (0, 0)),
        ],
        out_specs=pl.BlockSpec((1, tile_m, dh), lambda i, j: (i, j, 0)),
        out_shape=jax.ShapeDtypeStruct((b, n, dh), jnp.float32),
        compiler_params=pltpu.CompilerParams(
            dimension_semantics=("parallel", "parallel"),
        ),
    )(adj, keep, h, w, bias)


def _final_linear(x1, x2, x3, wlin, blin):
    b, n, e = x1.shape
    wt = wlin.T
    bias = blin.reshape(1, -1)
    return pl.pallas_call(
        _final_kernel,
        grid=(b,),
        in_specs=[
            pl.BlockSpec((1, n, e), lambda i: (i, 0, 0)),
            pl.BlockSpec((1, n, e), lambda i: (i, 0, 0)),
            pl.BlockSpec((1, n, e), lambda i: (i, 0, 0)),
            pl.BlockSpec(wt.shape, lambda i: (0, 0)),
            pl.BlockSpec((1, bias.shape[1]), lambda i: (0, 0)),
        ],
        out_specs=pl.BlockSpec((1, n, bias.shape[1]), lambda i: (i, 0, 0)),
        out_shape=jax.ShapeDtypeStruct((b, n, bias.shape[1]), jnp.float32),
        compiler_params=pltpu.CompilerParams(
            dimension_semantics=("parallel",),
        ),
    )(x1, x2, x3, wt, bias)


@jax.jit
def kernel(x, adj, W1, b1, W2, b2, W3, b3, Wlin, blin):
    b, n, _ = x.shape
    mask0 = (adj > 0).astype(jnp.int8)
    keep = _dilate(mask0)
    tile_m = 512
    x1 = _sage_layer(adj, keep, x, W1, b1.reshape(1, -1), tile_m)
    x2 = _sage_layer(adj, keep, x1, W2, b2.reshape(1, -1), tile_m)
    x3 = _sage_layer(adj, keep, x2, W3, b3.reshape(1, -1), tile_m)
    return _final_linear(x1, x2, x3, Wlin, blin)


# speedup vs baseline: 65.6634x; 1.3010x over previous
import functools

import jax
import jax.numpy as jnp
from jax import lax
from jax.experimental import pallas as pl
from jax.experimental.pallas import tpu as pltpu

_T = 10
_K = 2
_SUB = 16
_LANE = 128

_HIGH = lax.Precision.HIGHEST


def _dilate_kernel(m0_ref, keep_ref):
    keep_ref[...] = m0_ref[...].astype(jnp.float32)
    n_rows = m0_ref.shape[1]

    iota_h = lax.broadcasted_iota(jnp.int32, (_SUB, _LANE), 0)
    iota_l = lax.broadcasted_iota(jnp.int32, (_SUB, _LANE), 1)
    flat_iota = iota_h * _LANE + iota_l
    U = (
        lax.broadcasted_iota(jnp.int32, (_LANE, _LANE), 0)
        <= lax.broadcasted_iota(jnp.int32, (_LANE, _LANE), 1)
    ).astype(jnp.float32)
    Ls = (
        lax.broadcasted_iota(jnp.int32, (_SUB, _SUB), 1)
        < lax.broadcasted_iota(jnp.int32, (_SUB, _SUB), 0)
    ).astype(jnp.float32)
    ones_col = jnp.ones((_LANE, 1), jnp.float32)
    ones_row = jnp.ones((1, _SUB), jnp.float32)

    def apply_fix(c, col):
        ck = 1.0 - (
            (iota_h == col // _LANE) & (iota_l == col % _LANE)
        ).astype(jnp.float32)
        rc = keep_ref[0, pl.ds(c, 1)].reshape(_SUB, _LANE)
        keep_ref[0, pl.ds(c, 1)] = (rc * ck).reshape(1, _SUB, _LANE)

    def body(n, carry):
        a1, c1p, a2, c2p = carry

        @pl.when(a1 == 1)
        def _():
            apply_fix(c1p, n - 1)

        @pl.when(a2 == 1)
        def _():
            apply_fix(c2p, n - 1)

        row = keep_ref[0, pl.ds(n, 1)].reshape(_SUB, _LANE)
        pc = jnp.dot(row, U)
        rs = jnp.dot(row, ones_col)
        excl = jnp.dot(Ls, rs)
        m11 = jnp.dot(ones_row, rs)
        cs = pc + excl

        m = m11[0, 0].astype(jnp.int32)
        maskb = row > 0.0
        sf = jnp.where(m > _T, (m + (_K - 1)) // _K, 1)
        removable = m > 1
        small = m <= _T
        t1 = sf.astype(jnp.float32)
        t2 = (2 * sf).astype(jnp.float32)

        rem = maskb & removable & ((cs == t1) | (cs == t2) | small)
        keep_ref[0, pl.ds(n, 1)] = jnp.where(rem, 0.0, row).reshape(
            1, _SUB, _LANE
        )

        c1 = jnp.max(jnp.where(maskb & (cs == t1), flat_iota, -1))
        c2 = jnp.max(jnp.where(maskb & (cs == t2), flat_iota, -1))
        a1n = (removable & (sf <= m)).astype(jnp.int32)
        a2n = (removable & (2 * sf <= m)).astype(jnp.int32)

        @pl.when(removable & small & (m >= 3))
        def _():
            crs = [
                jnp.max(jnp.where(maskb & (cs == float(r)), flat_iota, -1))
                for r in range(3, _T + 1)
            ]
            for i, r in enumerate(range(3, _T + 1)):

                @pl.when(r <= m)
                def _(i=i):
                    apply_fix(crs[i], n)

        return (a1n, c1, a2n, c2)

    z = jnp.int32(0)
    a1, c1, a2, c2 = lax.fori_loop(0, n_rows, body, (z, z, z, z))

    @pl.when(a1 == 1)
    def _():
        apply_fix(c1, n_rows - 1)

    @pl.when(a2 == 1)
    def _():
        apply_fix(c2, n_rows - 1)


def _sage_kernel(adj_ref, keep_ref, h_ref, w_ref, b_ref, out_ref):
    ad = adj_ref[0] * keep_ref[0]
    t = jnp.dot(ad, h_ref[0], precision=_HIGH)
    deg = jnp.maximum(jnp.sum(ad, axis=1, keepdims=True), 1.0)
    t = t / deg
    y = jnp.dot(t, w_ref[...], precision=_HIGH) + b_ref[...]
    nrm = jnp.sqrt(jnp.sum(y * y, axis=1, keepdims=True))
    y = y / jnp.maximum(nrm, 1e-12)
    out_ref[0] = jnp.maximum(y, 0.0)


def _final_kernel(x1_ref, x2_ref, x3_ref, wt_ref, b_ref, out_ref):
    xc = jnp.concatenate([x1_ref[0], x2_ref[0], x3_ref[0]], axis=1)
    out_ref[0] = jnp.dot(xc, wt_ref[...], precision=_HIGH) + b_ref[...]


def _dilate(mask0):
    b, n = mask0.shape[0], mask0.shape[1]
    m4 = mask0.reshape(b, n, _SUB, _LANE)
    keep = pl.pallas_call(
        _dilate_kernel,
        grid=(b,),
        in_specs=[
            pl.BlockSpec((1, n, _SUB, _LANE), lambda i: (i, 0, 0, 0)),
        ],
        out_specs=pl.BlockSpec((1, n, _SUB, _LANE), lambda i: (i, 0, 0, 0)),
        out_shape=jax.ShapeDtypeStruct((b, n, _SUB, _LANE), jnp.float32),
        compiler_params=pltpu.CompilerParams(
            dimension_semantics=("arbitrary",),
        ),
    )(m4)
    return keep.reshape(b, n, n)


def _sage_layer(adj, keep, h, w, bias, tile_m):
    b, n, _ = adj.shape
    dh = w.shape[1]
    return pl.pallas_call(
        _sage_kernel,
        grid=(b, n // tile_m),
        in_specs=[
            pl.BlockSpec((1, tile_m, n), lambda i, j: (i, j, 0)),
            pl.BlockSpec((1, tile_m, n), lambda i, j: (i, j, 0)),
            pl.BlockSpec((1, n, h.shape[2]), lambda i, j: (i, 0, 0)),
            pl.BlockSpec(w.shape, lambda i, j: (0, 0)),
            pl.BlockSpec((1, dh), lambda i, j: (0, 0)),
        ],
        out_specs=pl.BlockSpec((1, tile_m, dh), lambda i, j: (i, j, 0)),
        out_shape=jax.ShapeDtypeStruct((b, n, dh), jnp.float32),
        compiler_params=pltpu.CompilerParams(
            dimension_semantics=("parallel", "parallel"),
        ),
    )(adj, keep, h, w, bias)


def _final_linear(x1, x2, x3, wlin, blin):
    b, n, e = x1.shape
    wt = wlin.T
    bias = blin.reshape(1, -1)
    return pl.pallas_call(
        _final_kernel,
        grid=(b,),
        in_specs=[
            pl.BlockSpec((1, n, e), lambda i: (i, 0, 0)),
            pl.BlockSpec((1, n, e), lambda i: (i, 0, 0)),
            pl.BlockSpec((1, n, e), lambda i: (i, 0, 0)),
            pl.BlockSpec(wt.shape, lambda i: (0, 0)),
            pl.BlockSpec((1, bias.shape[1]), lambda i: (0, 0)),
        ],
        out_specs=pl.BlockSpec((1, n, bias.shape[1]), lambda i: (i, 0, 0)),
        out_shape=jax.ShapeDtypeStruct((b, n, bias.shape[1]), jnp.float32),
        compiler_params=pltpu.CompilerParams(
            dimension_semantics=("parallel",),
        ),
    )(x1, x2, x3, wt, bias)


@jax.jit
def kernel(x, adj, W1, b1, W2, b2, W3, b3, Wlin, blin):
    b, n, _ = x.shape
    mask0 = (adj > 0).astype(jnp.int8)
    keep = _dilate(mask0)
    tile_m = 512
    x1 = _sage_layer(adj, keep, x, W1, b1.reshape(1, -1), tile_m)
    x2 = _sage_layer(adj, keep, x1, W2, b2.reshape(1, -1), tile_m)
    x3 = _sage_layer(adj, keep, x2, W3, b3.reshape(1, -1), tile_m)
    return _final_linear(x1, x2, x3, Wlin, blin)
